# Initial kernel scaffold; baseline (speedup 1.0000x reference)
import jax
import jax.numpy as jnp
from jax.experimental import pallas as pl


def kernel(x, edge_index, W1, b1, W2, b2):
    raise NotImplementedError("write your pallas kernel here")


# speedup vs baseline: 23.2332x; 23.2332x over previous
import functools

import jax
import jax.numpy as jnp
from jax import lax
from jax.experimental import pallas as pl
from jax.experimental.pallas import tpu as pltpu
from jax.experimental.pallas import tpu_sc as plsc

N_NODES = 10000
N_EDGES = 320000
D = 128

NC = 2
NS = 16
NW = NC * NS
EDGES_PER_TILE = N_EDGES // NW
EPC = 125
CHUNKS = EDGES_PER_TILE // EPC
RPS = 624
TAIL = N_NODES - NS * RPS
TAIL0 = NS * RPS

_sc_mesh = plsc.VectorSubcoreMesh(core_axis_name="c", subcore_axis_name="s")


@functools.partial(
    pl.kernel,
    out_type=[jax.ShapeDtypeStruct((N_NODES,), jnp.float32),
              jax.ShapeDtypeStruct((N_NODES,), jnp.float32)],
    mesh=_sc_mesh,
    scratch_types=[
        pltpu.VMEM((CHUNKS, EPC), jnp.int32),
        pltpu.VMEM((128,), jnp.float32),
        pltpu.VMEM((RPS,), jnp.float32),
        pltpu.VMEM_SHARED((N_NODES,), jnp.float32),
    ],
)
def _sc_degree(dst_hbm, zeros_hbm, out0_hbm, out1_hbm, dst_v, ones_v,
               bounce_v, deg_s):
    c = lax.axis_index("c")
    s = lax.axis_index("s")
    wid = s * NC + c
    r0 = s * RPS
    pltpu.sync_copy(zeros_hbm.at[pl.ds(r0, RPS)], bounce_v)
    pltpu.sync_copy(bounce_v, deg_s.at[pl.ds(r0, RPS)])

    @pl.when(s == 0)
    def _():
        pltpu.sync_copy(zeros_hbm.at[pl.ds(TAIL0, TAIL)],
                        bounce_v.at[pl.ds(0, TAIL)])
        pltpu.sync_copy(bounce_v.at[pl.ds(0, TAIL)],
                        deg_s.at[pl.ds(TAIL0, TAIL)])

    for k in range(8):
        ones_v[pl.ds(k * 16, 16)] = jnp.full((16,), 1.0, jnp.float32)
    pltpu.sync_copy(dst_hbm.at[wid], dst_v)
    plsc.subcore_barrier()

    def body(j, carry):
        pltpu.sync_copy(ones_v.at[pl.ds(0, EPC)], deg_s.at[dst_v.at[j]],
                        add=True)
        return carry

    lax.fori_loop(0, CHUNKS, body, 0)
    plsc.subcore_barrier()

    pltpu.sync_copy(deg_s.at[pl.ds(r0, RPS)], bounce_v)

    @pl.when(c == 0)
    def _():
        pltpu.sync_copy(bounce_v, out0_hbm.at[pl.ds(r0, RPS)])

    @pl.when(c == 1)
    def _():
        pltpu.sync_copy(bounce_v, out1_hbm.at[pl.ds(r0, RPS)])

    @pl.when(s == 0)
    def _():
        pltpu.sync_copy(deg_s.at[pl.ds(TAIL0, TAIL)],
                        bounce_v.at[pl.ds(0, TAIL)])

        @pl.when(c == 0)
        def _():
            pltpu.sync_copy(bounce_v.at[pl.ds(0, TAIL)],
                            out0_hbm.at[pl.ds(TAIL0, TAIL)])

        @pl.when(c == 1)
        def _():
            pltpu.sync_copy(bounce_v.at[pl.ds(0, TAIL)],
                            out1_hbm.at[pl.ds(TAIL0, TAIL)])


@functools.partial(
    pl.kernel,
    out_type=jax.ShapeDtypeStruct((NC, N_NODES, D), jnp.float32),
    mesh=_sc_mesh,
    scratch_types=[
        pltpu.VMEM((CHUNKS, EPC), jnp.int32),
        pltpu.VMEM((CHUNKS, EPC), jnp.int32),
        pltpu.VMEM((EPC, D), jnp.float32),
        pltpu.VMEM_SHARED((N_NODES, D), jnp.float32),
    ],
)
def _sc_aggregate(src_hbm, dst_hbm, hs_hbm, zeros_hbm, out_hbm,
                  src_v, dst_v, rows_v, agg_s):
    c = lax.axis_index("c")
    s = lax.axis_index("s")
    wid = s * NC + c
    r0 = s * RPS
    pltpu.sync_copy(zeros_hbm.at[pl.ds(r0, RPS)], agg_s.at[pl.ds(r0, RPS)])

    @pl.when(s == 0)
    def _():
        pltpu.sync_copy(zeros_hbm.at[pl.ds(TAIL0, TAIL)],
                        agg_s.at[pl.ds(TAIL0, TAIL)])

    pltpu.sync_copy(src_hbm.at[wid], src_v)
    pltpu.sync_copy(dst_hbm.at[wid], dst_v)
    plsc.subcore_barrier()

    def body(j, carry):
        pltpu.sync_copy(hs_hbm.at[src_v.at[j]], rows_v)
        pltpu.sync_copy(rows_v, agg_s.at[dst_v.at[j]], add=True)
        return carry

    lax.fori_loop(0, CHUNKS, body, 0)
    plsc.subcore_barrier()
    pltpu.sync_copy(agg_s.at[pl.ds(r0, RPS)], out_hbm.at[c, pl.ds(r0, RPS)])

    @pl.when(s == 0)
    def _():
        pltpu.sync_copy(agg_s.at[pl.ds(TAIL0, TAIL)],
                        out_hbm.at[c, pl.ds(TAIL0, TAIL)])


_RB = 2000
_GRID = N_NODES // _RB


def _tc1_body(d0_ref, d1_ref, x_ref, w_ref, hs_ref, dinv_ref):
    dinv = lax.rsqrt(d0_ref[0, 0, :] + d1_ref[0, 0, :] + 1.0)
    dinv_ref[0, 0, :] = dinv
    h = jnp.dot(x_ref[...], w_ref[...], preferred_element_type=jnp.float32)
    hs_ref[...] = h * dinv[:, None]


def _tc2_body(dinv_ref, p_ref, hs_ref, b_ref, w_ref, out_ref):
    dinv = dinv_ref[0, 0, :][:, None]
    z = (p_ref[0] + p_ref[1] + hs_ref[...]) * dinv + b_ref[...]
    a = jnp.maximum(z, 0.0)
    out_ref[...] = jnp.dot(a, w_ref[...],
                           preferred_element_type=jnp.float32) * dinv


def _tc3_body(dinv_ref, q_ref, hs_ref, b_ref, out_ref):
    dinv = dinv_ref[0, 0, :][:, None]
    z = (q_ref[0] + q_ref[1] + hs_ref[...]) * dinv + b_ref[...]
    z = jnp.maximum(z, 0.0)
    m = jnp.max(z, axis=1, keepdims=True)
    e = jnp.exp(z - m)
    lse = jnp.log(jnp.sum(e, axis=1, keepdims=True))
    out_ref[...] = z - m - lse


_vec_spec = pl.BlockSpec((1, 1, _RB), lambda i: (i, 0, 0))
_row_spec = pl.BlockSpec((_RB, D), lambda i: (i, 0))
_p_spec = pl.BlockSpec((NC, _RB, D), lambda i: (0, i, 0))
_w_spec = pl.BlockSpec((D, D), lambda i: (0, 0))
_b_spec = pl.BlockSpec((1, D), lambda i: (0, 0))

_tc1 = pl.pallas_call(
    _tc1_body,
    grid=(_GRID,),
    in_specs=[_vec_spec, _vec_spec, _row_spec, _w_spec],
    out_specs=[_row_spec, _vec_spec],
    out_shape=[jax.ShapeDtypeStruct((N_NODES, D), jnp.float32),
               jax.ShapeDtypeStruct((_GRID, 1, _RB), jnp.float32)],
)

_tc2 = pl.pallas_call(
    _tc2_body,
    grid=(_GRID,),
    in_specs=[_vec_spec, _p_spec, _row_spec, _b_spec, _w_spec],
    out_specs=_row_spec,
    out_shape=jax.ShapeDtypeStruct((N_NODES, D), jnp.float32),
)

_tc3 = pl.pallas_call(
    _tc3_body,
    grid=(_GRID,),
    in_specs=[_vec_spec, _p_spec, _row_spec, _b_spec],
    out_specs=_row_spec,
    out_shape=jax.ShapeDtypeStruct((N_NODES, D), jnp.float32),
)


def kernel(x, edge_index, W1, b1, W2, b2):
    src = edge_index[0].astype(jnp.int32).reshape(NW, CHUNKS, EPC)
    dst = edge_index[1].astype(jnp.int32).reshape(NW, CHUNKS, EPC)
    zeros_deg = jnp.zeros((N_NODES,), jnp.float32)
    zeros_row = jnp.zeros((N_NODES, D), jnp.float32)

    deg0, deg1 = _sc_degree(dst, zeros_deg)
    deg0 = deg0.reshape(_GRID, 1, _RB)
    deg1 = deg1.reshape(_GRID, 1, _RB)
    hs1, dinv = _tc1(deg0, deg1, x, W1)
    p1 = _sc_aggregate(src, dst, hs1, zeros_row)
    hs2 = _tc2(dinv, p1, hs1, b1.reshape(1, D), W2)
    p2 = _sc_aggregate(src, dst, hs2, zeros_row)
    return _tc3(dinv, p2, hs2, b2.reshape(1, D))


# speedup vs baseline: 32.3211x; 1.3912x over previous
import functools

import jax
import jax.numpy as jnp
from jax import lax
from jax.experimental import pallas as pl
from jax.experimental.pallas import tpu as pltpu
from jax.experimental.pallas import tpu_sc as plsc

N_NODES = 10000
N_EDGES = 320000
D = 128

NC = 2
NS = 16
NW = NC * NS
EDGES_PER_TILE = N_EDGES // NW
EPC = 100
CHUNKS = EDGES_PER_TILE // EPC
RPS = 624
TAIL = N_NODES - NS * RPS
TAIL0 = NS * RPS

_sc_mesh = plsc.VectorSubcoreMesh(core_axis_name="c", subcore_axis_name="s")


@functools.partial(
    pl.kernel,
    out_type=[jax.ShapeDtypeStruct((N_NODES,), jnp.float32),
              jax.ShapeDtypeStruct((N_NODES,), jnp.float32)],
    mesh=_sc_mesh,
    scratch_types=[
        pltpu.VMEM((CHUNKS, EPC), jnp.int32),
        pltpu.VMEM((128,), jnp.float32),
        pltpu.VMEM((RPS,), jnp.float32),
        pltpu.VMEM_SHARED((N_NODES,), jnp.float32),
    ],
)
def _sc_degree(dst_hbm, zeros_hbm, out0_hbm, out1_hbm, dst_v, ones_v,
               bounce_v, deg_s):
    c = lax.axis_index("c")
    s = lax.axis_index("s")
    wid = s * NC + c
    r0 = s * RPS
    pltpu.sync_copy(zeros_hbm.at[pl.ds(r0, RPS)], bounce_v)
    pltpu.sync_copy(bounce_v, deg_s.at[pl.ds(r0, RPS)])

    @pl.when(s == 0)
    def _():
        pltpu.sync_copy(zeros_hbm.at[pl.ds(TAIL0, TAIL)],
                        bounce_v.at[pl.ds(0, TAIL)])
        pltpu.sync_copy(bounce_v.at[pl.ds(0, TAIL)],
                        deg_s.at[pl.ds(TAIL0, TAIL)])

    for k in range(8):
        ones_v[pl.ds(k * 16, 16)] = jnp.full((16,), 1.0, jnp.float32)
    for h in range(2):
        pltpu.sync_copy(dst_hbm.at[wid, h],
                        dst_v.at[pl.ds(h * (CHUNKS // 2), CHUNKS // 2)])
    plsc.subcore_barrier()

    def body(j, carry):
        pltpu.sync_copy(ones_v.at[pl.ds(0, EPC)], deg_s.at[dst_v.at[j]],
                        add=True)
        return carry

    lax.fori_loop(0, CHUNKS, body, 0)
    plsc.subcore_barrier()

    pltpu.sync_copy(deg_s.at[pl.ds(r0, RPS)], bounce_v)

    @pl.when(c == 0)
    def _():
        pltpu.sync_copy(bounce_v, out0_hbm.at[pl.ds(r0, RPS)])

    @pl.when(c == 1)
    def _():
        pltpu.sync_copy(bounce_v, out1_hbm.at[pl.ds(r0, RPS)])

    @pl.when(s == 0)
    def _():
        pltpu.sync_copy(deg_s.at[pl.ds(TAIL0, TAIL)],
                        bounce_v.at[pl.ds(0, TAIL)])

        @pl.when(c == 0)
        def _():
            pltpu.sync_copy(bounce_v.at[pl.ds(0, TAIL)],
                            out0_hbm.at[pl.ds(TAIL0, TAIL)])

        @pl.when(c == 1)
        def _():
            pltpu.sync_copy(bounce_v.at[pl.ds(0, TAIL)],
                            out1_hbm.at[pl.ds(TAIL0, TAIL)])


@functools.partial(
    pl.kernel,
    out_type=jax.ShapeDtypeStruct((NC, N_NODES, D), jnp.float32),
    mesh=_sc_mesh,
    scratch_types=[
        pltpu.VMEM_SHARED((N_NODES, D), jnp.float32),
        pltpu.VMEM((CHUNKS // 2, EPC), jnp.int32),
        pltpu.VMEM((CHUNKS // 2, EPC), jnp.int32),
        pltpu.VMEM((2, EPC, D), jnp.float32),
        pltpu.SemaphoreType.DMA,
        pltpu.SemaphoreType.DMA,
    ],
)
def _sc_aggregate(src_hbm, dst_hbm, hs_hbm, zeros_hbm, out_hbm,
                  agg_s, src_v, dst_v, rows_v, sem0, sem1):
    c = lax.axis_index("c")
    s = lax.axis_index("s")
    wid = s * NC + c
    r0 = s * RPS
    pltpu.sync_copy(zeros_hbm.at[pl.ds(r0, RPS)], agg_s.at[pl.ds(r0, RPS)])

    @pl.when(s == 0)
    def _():
        pltpu.sync_copy(zeros_hbm.at[pl.ds(TAIL0, TAIL)],
                        agg_s.at[pl.ds(TAIL0, TAIL)])

    plsc.subcore_barrier()

    sems = (sem0, sem1)
    HC = CHUNKS // 2
    for h in range(2):
        pltpu.sync_copy(src_hbm.at[wid, h], src_v)
        pltpu.sync_copy(dst_hbm.at[wid, h], dst_v)
        for b in range(2):
            pltpu.async_copy(hs_hbm.at[src_v.at[b]], rows_v.at[b], sems[b])

        def body(k, carry):
            for b in range(2):
                j = 2 * k + b
                pltpu.make_async_copy(hs_hbm.at[src_v.at[j]], rows_v.at[b],
                                      sems[b]).wait()
                pltpu.sync_copy(rows_v.at[b], agg_s.at[dst_v.at[j]],
                                add=True)

                @pl.when(k < HC // 2 - 1)
                def _():
                    pltpu.async_copy(hs_hbm.at[src_v.at[j + 2]],
                                     rows_v.at[b], sems[b])

            return carry

        lax.fori_loop(0, HC // 2, body, 0)

    plsc.subcore_barrier()
    pltpu.sync_copy(agg_s.at[pl.ds(r0, RPS)], out_hbm.at[c, pl.ds(r0, RPS)])

    @pl.when(s == 0)
    def _():
        pltpu.sync_copy(agg_s.at[pl.ds(TAIL0, TAIL)],
                        out_hbm.at[c, pl.ds(TAIL0, TAIL)])


_RB = 2000
_GRID = N_NODES // _RB


def _tc1_body(d0_ref, d1_ref, x_ref, w_ref, hs_ref, dinv_ref):
    dinv = lax.rsqrt(d0_ref[0, 0, :] + d1_ref[0, 0, :] + 1.0)
    dinv_ref[0, 0, :] = dinv
    h = jnp.dot(x_ref[...], w_ref[...], preferred_element_type=jnp.float32)
    hs_ref[...] = h * dinv[:, None]


def _tc2_body(dinv_ref, p_ref, hs_ref, b_ref, w_ref, out_ref):
    dinv = dinv_ref[0, 0, :][:, None]
    z = (p_ref[0] + p_ref[1] + hs_ref[...]) * dinv + b_ref[...]
    a = jnp.maximum(z, 0.0)
    out_ref[...] = jnp.dot(a, w_ref[...],
                           preferred_element_type=jnp.float32) * dinv


def _tc3_body(dinv_ref, q_ref, hs_ref, b_ref, out_ref):
    dinv = dinv_ref[0, 0, :][:, None]
    z = (q_ref[0] + q_ref[1] + hs_ref[...]) * dinv + b_ref[...]
    z = jnp.maximum(z, 0.0)
    m = jnp.max(z, axis=1, keepdims=True)
    e = jnp.exp(z - m)
    lse = jnp.log(jnp.sum(e, axis=1, keepdims=True))
    out_ref[...] = z - m - lse


_vec_spec = pl.BlockSpec((1, 1, _RB), lambda i: (i, 0, 0))
_row_spec = pl.BlockSpec((_RB, D), lambda i: (i, 0))
_p_spec = pl.BlockSpec((NC, _RB, D), lambda i: (0, i, 0))
_w_spec = pl.BlockSpec((D, D), lambda i: (0, 0))
_b_spec = pl.BlockSpec((1, D), lambda i: (0, 0))

_tc1 = pl.pallas_call(
    _tc1_body,
    grid=(_GRID,),
    in_specs=[_vec_spec, _vec_spec, _row_spec, _w_spec],
    out_specs=[_row_spec, _vec_spec],
    out_shape=[jax.ShapeDtypeStruct((N_NODES, D), jnp.float32),
               jax.ShapeDtypeStruct((_GRID, 1, _RB), jnp.float32)],
)

_tc2 = pl.pallas_call(
    _tc2_body,
    grid=(_GRID,),
    in_specs=[_vec_spec, _p_spec, _row_spec, _b_spec, _w_spec],
    out_specs=_row_spec,
    out_shape=jax.ShapeDtypeStruct((N_NODES, D), jnp.float32),
)

_tc3 = pl.pallas_call(
    _tc3_body,
    grid=(_GRID,),
    in_specs=[_vec_spec, _p_spec, _row_spec, _b_spec],
    out_specs=_row_spec,
    out_shape=jax.ShapeDtypeStruct((N_NODES, D), jnp.float32),
)


def kernel(x, edge_index, W1, b1, W2, b2):
    src = edge_index[0].astype(jnp.int32).reshape(NW, 2, CHUNKS // 2, EPC)
    dst = edge_index[1].astype(jnp.int32).reshape(NW, 2, CHUNKS // 2, EPC)
    zeros_deg = jnp.zeros((N_NODES,), jnp.float32)
    zeros_row = jnp.zeros((N_NODES, D), jnp.float32)

    deg0, deg1 = _sc_degree(dst, zeros_deg)
    deg0 = deg0.reshape(_GRID, 1, _RB)
    deg1 = deg1.reshape(_GRID, 1, _RB)
    hs1, dinv = _tc1(deg0, deg1, x, W1)
    p1 = _sc_aggregate(src, dst, hs1, zeros_row)
    hs2 = _tc2(dinv, p1, hs1, b1.reshape(1, D), W2)
    p2 = _sc_aggregate(src, dst, hs2, zeros_row)
    return _tc3(dinv, p2, hs2, b2.reshape(1, D))
